# baseline (speedup 1.0000x reference)
import functools

import jax
import jax.numpy as jnp
from jax.experimental import pallas as pl
from jax.experimental.pallas import tpu as pltpu

_Q = 0.5
_NC = 16
_SLOTS = 26
_LAG = 8


def _fused_body(k_lo, k_hi, B, N, D, x_hbm, o_hbm, buf, nrm_k, nrm_c,
                sem_in, sem_out):
    R = N // _NC
    S = R // 128
    G = B * _NC

    def in_copy(g):
        b, c = divmod(g, _NC)
        return pltpu.make_async_copy(
            x_hbm.at[b, pl.ds(c * R, R), :],
            buf.at[g % _SLOTS],
            sem_in.at[g % _SLOTS])

    def out_copy(g):
        b, c = divmod(g, _NC)
        return pltpu.make_async_copy(
            buf.at[g % _SLOTS],
            o_hbm.at[b, pl.ds(c * R, R), :],
            sem_out.at[g % _SLOTS])

    col = jax.lax.broadcasted_iota(jnp.int32, (1, 2), 1)
    targets = jnp.where(col == 0, k_lo + 1, k_hi + 1)

    for g in range(_SLOTS):
        in_copy(g).start()

    for b in range(B):
        for c in range(_NC):
            g = b * _NC + c
            in_copy(g).wait()
            xb = buf[g % _SLOTS]
            sq = xb * xb
            nrm_k[pl.ds(c * R, R), :] = jnp.sqrt(
                jnp.sum(sq, axis=1, keepdims=True))
            nrm_c[pl.ds(c * S, S), :] = jnp.sqrt(
                jnp.sum(sq.reshape(S, 128, D), axis=2))

        bits = jax.lax.bitcast_convert_type(nrm_c[...], jnp.int32)

        def srch(_, lh, bits=bits):
            lo, hi = lh
            mid = lo + (hi - lo) // 2
            cnt = jnp.sum((bits[None, None, :, :] <= mid[:, :, None, None])
                          .astype(jnp.int32), axis=(2, 3))
            pred = cnt >= targets
            return jnp.where(pred, lo, mid + 1), jnp.where(pred, mid, hi)

        lo0 = jnp.zeros((1, 2), jnp.int32)
        hi0 = jnp.full((1, 2), 0x7F800000, jnp.int32)
        lo, _ = jax.lax.fori_loop(0, 31, srch, (lo0, hi0))
        v = jax.lax.bitcast_convert_type(lo, jnp.float32)
        thres = v[:, 0:1] * 0.5 + v[:, 1:2] * 0.5

        for c in range(_NC):
            g = b * _NC + c
            m = (~(nrm_k[pl.ds(c * R, R), :] < thres)).astype(jnp.float32)
            buf[g % _SLOTS] = buf[g % _SLOTS] * m
            out_copy(g).start()
            h = g + _SLOTS - _LAG
            if g - _LAG >= 0 and h < G:
                out_copy(g - _LAG).wait()
                in_copy(h).start()

    for g in range(max(G - _SLOTS, 0), G):
        out_copy(g).wait()


def kernel(x):
    B, N, D = x.shape
    k_lo = int(_Q * (N - 1))
    k_hi = k_lo + (1 if _Q * (N - 1) != k_lo else 0)
    R = N // _NC
    return pl.pallas_call(
        functools.partial(_fused_body, k_lo, k_hi, B, N, D),
        in_specs=[pl.BlockSpec(memory_space=pl.ANY)],
        out_specs=pl.BlockSpec(memory_space=pl.ANY),
        out_shape=jax.ShapeDtypeStruct((B, N, D), x.dtype),
        scratch_shapes=[
            pltpu.VMEM((_SLOTS, R, D), jnp.float32),
            pltpu.VMEM((N, 1), jnp.float32),
            pltpu.VMEM((N // 128, 128), jnp.float32),
            pltpu.SemaphoreType.DMA((_SLOTS,)),
            pltpu.SemaphoreType.DMA((_SLOTS,)),
        ],
    )(x)


# speedup vs baseline: 1.0110x; 1.0110x over previous
import functools

import jax
import jax.numpy as jnp
from jax.experimental import pallas as pl
from jax.experimental.pallas import tpu as pltpu

_Q = 0.5
_NC = 16
_SLOTS = 26
_LAG = 4
_HOLD = 4


def _fused_body(k_lo, k_hi, B, N, D, x_hbm, o_hbm, buf, nrm_k, nrm_c,
                sem_in, sem_out):
    R = N // _NC
    S = R // 128
    G = B * _NC

    def in_copy(g):
        b, c = divmod(g, _NC)
        return pltpu.make_async_copy(
            x_hbm.at[b, pl.ds(c * R, R), :],
            buf.at[g % _SLOTS],
            sem_in.at[g % _SLOTS])

    def out_copy(g):
        b, c = divmod(g, _NC)
        return pltpu.make_async_copy(
            buf.at[g % _SLOTS],
            o_hbm.at[b, pl.ds(c * R, R), :],
            sem_out.at[g % _SLOTS])

    col = jax.lax.broadcasted_iota(jnp.int32, (1, 2), 1)
    targets = jnp.where(col == 0, k_lo + 1, k_hi + 1)

    for g in range(_SLOTS):
        in_copy(g).start()

    for b in range(B):
        for c in range(_NC):
            g = b * _NC + c
            in_copy(g).wait()
            xb = buf[g % _SLOTS]
            sq = xb * xb
            nrm_k[pl.ds(c * R, R), :] = jnp.sqrt(
                jnp.sum(sq, axis=1, keepdims=True))
            nrm_c[pl.ds(c * S, S), :] = jnp.sqrt(
                jnp.sum(sq.reshape(S, 128, D), axis=2))

        if b > 0:
            for c in range(_NC - _HOLD, _NC):
                out_copy((b - 1) * _NC + c).start()

        bits = jax.lax.bitcast_convert_type(nrm_c[...], jnp.int32)

        def srch(_, lh, bits=bits):
            lo, hi = lh
            mid = lo + (hi - lo) // 2
            cnt = jnp.sum((bits[None, None, :, :] <= mid[:, :, None, None])
                          .astype(jnp.int32), axis=(2, 3))
            pred = cnt >= targets
            return jnp.where(pred, lo, mid + 1), jnp.where(pred, mid, hi)

        lo0 = jnp.zeros((1, 2), jnp.int32)
        hi0 = jnp.full((1, 2), 0x7F800000, jnp.int32)
        lo, _ = jax.lax.fori_loop(0, 31, srch, (lo0, hi0))
        v = jax.lax.bitcast_convert_type(lo, jnp.float32)
        thres = v[:, 0:1] * 0.5 + v[:, 1:2] * 0.5

        for c in range(_NC):
            g = b * _NC + c
            m = (~(nrm_k[pl.ds(c * R, R), :] < thres)).astype(jnp.float32)
            buf[g % _SLOTS] = buf[g % _SLOTS] * m
            if c < _NC - _HOLD or b == B - 1:
                out_copy(g).start()
            h = g + _SLOTS - _LAG
            if g - _LAG >= 0 and h < G:
                out_copy(g - _LAG).wait()
                in_copy(h).start()

    for g in range(max(G - _SLOTS, 0), G):
        out_copy(g).wait()


def kernel(x):
    B, N, D = x.shape
    k_lo = int(_Q * (N - 1))
    k_hi = k_lo + (1 if _Q * (N - 1) != k_lo else 0)
    R = N // _NC
    return pl.pallas_call(
        functools.partial(_fused_body, k_lo, k_hi, B, N, D),
        in_specs=[pl.BlockSpec(memory_space=pl.ANY)],
        out_specs=pl.BlockSpec(memory_space=pl.ANY),
        out_shape=jax.ShapeDtypeStruct((B, N, D), x.dtype),
        scratch_shapes=[
            pltpu.VMEM((_SLOTS, R, D), jnp.float32),
            pltpu.VMEM((N, 1), jnp.float32),
            pltpu.VMEM((N // 128, 128), jnp.float32),
            pltpu.SemaphoreType.DMA((_SLOTS,)),
            pltpu.SemaphoreType.DMA((_SLOTS,)),
        ],
    )(x)


# speedup vs baseline: 1.0398x; 1.0285x over previous
import functools

import jax
import jax.numpy as jnp
from jax.experimental import pallas as pl
from jax.experimental.pallas import tpu as pltpu

_Q = 0.5
_NC = 16
_SLOTS = 26


def _fused_body(k_lo, k_hi, B, N, D, x_hbm, o_hbm, buf, nrm_k, nrm_c,
                sem_in, sem_out):
    R = N // _NC
    S = R // 128
    G = B * _NC

    def in_copy(g):
        b, c = divmod(g, _NC)
        return pltpu.make_async_copy(
            x_hbm.at[b, pl.ds(c * R, R), :],
            buf.at[g % _SLOTS],
            sem_in.at[g % _SLOTS])

    def out_copy(g):
        b, c = divmod(g, _NC)
        return pltpu.make_async_copy(
            buf.at[g % _SLOTS],
            o_hbm.at[b, pl.ds(c * R, R), :],
            sem_out.at[g % _SLOTS])

    col = jax.lax.broadcasted_iota(jnp.int32, (1, 2), 1)
    targets = jnp.where(col == 0, k_lo + 1, k_hi + 1)

    for g in range(min(_SLOTS, G)):
        in_copy(g).start()

    for b in range(B):
        for c in range(_NC):
            g = b * _NC + c
            in_copy(g).wait()
            xb = buf[g % _SLOTS]
            sq = xb * xb
            nrm_k[pl.ds(c * R, R), :] = jnp.sqrt(
                jnp.sum(sq, axis=1, keepdims=True))
            nrm_c[pl.ds(c * S, S), :] = jnp.sqrt(
                jnp.sum(sq.reshape(S, 128, D), axis=2))
            h = g + _SLOTS - _NC
            if g >= _NC and h < G:
                out_copy(g - _NC).wait()
                in_copy(h).start()

        bits = jax.lax.bitcast_convert_type(nrm_c[...], jnp.int32)

        def srch(_, lh, bits=bits):
            lo, hi = lh
            mid = lo + (hi - lo) // 2
            cnt = jnp.sum((bits[None, None, :, :] <= mid[:, :, None, None])
                          .astype(jnp.int32), axis=(2, 3))
            pred = cnt >= targets
            return jnp.where(pred, lo, mid + 1), jnp.where(pred, mid, hi)

        lo0 = jnp.zeros((1, 2), jnp.int32)
        hi0 = jnp.full((1, 2), 0x7F800000, jnp.int32)
        lo, _ = jax.lax.fori_loop(0, 31, srch, (lo0, hi0))
        v = jax.lax.bitcast_convert_type(lo, jnp.float32)
        thres = v[:, 0:1] * 0.5 + v[:, 1:2] * 0.5

        for c in range(_NC):
            g = b * _NC + c
            m = (~(nrm_k[pl.ds(c * R, R), :] < thres)).astype(jnp.float32)
            buf[g % _SLOTS] = buf[g % _SLOTS] * m
            out_copy(g).start()

    for g in range(max(G - _SLOTS, 0), G):
        out_copy(g).wait()


def kernel(x):
    B, N, D = x.shape
    k_lo = int(_Q * (N - 1))
    k_hi = k_lo + (1 if _Q * (N - 1) != k_lo else 0)
    R = N // _NC
    return pl.pallas_call(
        functools.partial(_fused_body, k_lo, k_hi, B, N, D),
        in_specs=[pl.BlockSpec(memory_space=pl.ANY)],
        out_specs=pl.BlockSpec(memory_space=pl.ANY),
        out_shape=jax.ShapeDtypeStruct((B, N, D), x.dtype),
        scratch_shapes=[
            pltpu.VMEM((_SLOTS, R, D), jnp.float32),
            pltpu.VMEM((N, 1), jnp.float32),
            pltpu.VMEM((N // 128, 128), jnp.float32),
            pltpu.SemaphoreType.DMA((_SLOTS,)),
            pltpu.SemaphoreType.DMA((_SLOTS,)),
        ],
    )(x)


# speedup vs baseline: 1.0409x; 1.0011x over previous
import functools

import jax
import jax.numpy as jnp
from jax.experimental import pallas as pl
from jax.experimental.pallas import tpu as pltpu

_Q = 0.5
_NC = 16
_SLOTS = 26


def _fused_body(k_lo, k_hi, B, N, D, x_hbm, o_hbm, buf, nrm_k, nrm_c,
                sem_in, sem_out):
    R = N // _NC
    S = R // 128
    G = B * _NC

    def in_copy(g):
        b, c = divmod(g, _NC)
        return pltpu.make_async_copy(
            x_hbm.at[b, pl.ds(c * R, R), :],
            buf.at[g % _SLOTS],
            sem_in.at[g % _SLOTS])

    def out_copy(g):
        b, c = divmod(g, _NC)
        return pltpu.make_async_copy(
            buf.at[g % _SLOTS],
            o_hbm.at[b, pl.ds(c * R, R), :],
            sem_out.at[g % _SLOTS])

    col = jax.lax.broadcasted_iota(jnp.int32, (1, 2), 1)
    targets = jnp.where(col == 0, k_lo + 1, k_hi + 1)

    for g in range(min(_SLOTS, G)):
        in_copy(g).start()

    for b in range(B):
        for c in range(_NC):
            g = b * _NC + c
            in_copy(g).wait()
            xb = buf[g % _SLOTS]
            sq = xb * xb
            nrm_k[pl.ds(c * R, R), :] = jnp.sqrt(
                jnp.sum(sq, axis=1, keepdims=True))
            nrm_c[pl.ds(c * S, S), :] = jnp.sqrt(
                jnp.sum(sq.reshape(S, 128, D), axis=2))
            h = g + _SLOTS - _NC
            if g >= _NC and h < G:
                out_copy(g - _NC).wait()
                in_copy(h).start()

        bits = jax.lax.bitcast_convert_type(nrm_c[...], jnp.int32)

        def srch(_, lh, bits=bits):
            lo, hi = lh
            delta = hi - lo
            q, r = delta // 8, delta % 8
            jj = jax.lax.broadcasted_iota(jnp.int32, (1, 2, 7), 2) + 1
            m = (lo[:, :, None] + q[:, :, None] * jj
                 + (r[:, :, None] * jj) // 8)
            cnt = jnp.sum((bits[None, None, None, :, :]
                           <= m[:, :, :, None, None]).astype(jnp.int32),
                          axis=(3, 4))
            t3 = targets[:, :, None]
            big = jnp.full_like(m, 0x7F800001)
            new_lo = jnp.maximum(lo, jnp.max(
                jnp.where(cnt < t3, m + 1, 0), axis=2))
            new_hi = jnp.minimum(hi, jnp.min(
                jnp.where(cnt >= t3, m, big), axis=2))
            return new_lo, new_hi

        lo0 = jnp.zeros((1, 2), jnp.int32)
        hi0 = jnp.full((1, 2), 0x7F800000, jnp.int32)
        lo, _ = jax.lax.fori_loop(0, 13, srch, (lo0, hi0))
        v = jax.lax.bitcast_convert_type(lo, jnp.float32)
        thres = v[:, 0:1] * 0.5 + v[:, 1:2] * 0.5

        for c in range(_NC):
            g = b * _NC + c
            m = (~(nrm_k[pl.ds(c * R, R), :] < thres)).astype(jnp.float32)
            buf[g % _SLOTS] = buf[g % _SLOTS] * m
            out_copy(g).start()

    for g in range(max(G - _SLOTS, 0), G):
        out_copy(g).wait()


def kernel(x):
    B, N, D = x.shape
    k_lo = int(_Q * (N - 1))
    k_hi = k_lo + (1 if _Q * (N - 1) != k_lo else 0)
    R = N // _NC
    return pl.pallas_call(
        functools.partial(_fused_body, k_lo, k_hi, B, N, D),
        in_specs=[pl.BlockSpec(memory_space=pl.ANY)],
        out_specs=pl.BlockSpec(memory_space=pl.ANY),
        out_shape=jax.ShapeDtypeStruct((B, N, D), x.dtype),
        scratch_shapes=[
            pltpu.VMEM((_SLOTS, R, D), jnp.float32),
            pltpu.VMEM((N, 1), jnp.float32),
            pltpu.VMEM((N // 128, 128), jnp.float32),
            pltpu.SemaphoreType.DMA((_SLOTS,)),
            pltpu.SemaphoreType.DMA((_SLOTS,)),
        ],
    )(x)


# speedup vs baseline: 1.0436x; 1.0026x over previous
import functools

import jax
import jax.numpy as jnp
from jax.experimental import pallas as pl
from jax.experimental.pallas import tpu as pltpu

_Q = 0.5
_NC = 16
_SLOTS = 26
_HIN = 3


def _fused_body(k_lo, k_hi, B, N, D, x_hbm, o_hbm, buf, nrm_k, nrm_c,
                sem_in, sem_out):
    R = N // _NC
    S = R // 128
    G = B * _NC

    def in_copy(g):
        b, c = divmod(g, _NC)
        return pltpu.make_async_copy(
            x_hbm.at[b, pl.ds(c * R, R), :],
            buf.at[g % _SLOTS],
            sem_in.at[g % _SLOTS])

    def out_copy(g):
        b, c = divmod(g, _NC)
        return pltpu.make_async_copy(
            buf.at[g % _SLOTS],
            o_hbm.at[b, pl.ds(c * R, R), :],
            sem_out.at[g % _SLOTS])

    col = jax.lax.broadcasted_iota(jnp.int32, (1, 2), 1)
    targets = jnp.where(col == 0, k_lo + 1, k_hi + 1)

    for g in range(min(_SLOTS, G)):
        in_copy(g).start()

    for b in range(B):
        for c in range(_NC):
            g = b * _NC + c
            in_copy(g).wait()
            xb = buf[g % _SLOTS]
            sq = xb * xb
            nrm_k[pl.ds(c * R, R), :] = jnp.sqrt(
                jnp.sum(sq, axis=1, keepdims=True))
            nrm_c[pl.ds(c * S, S), :] = jnp.sqrt(
                jnp.sum(sq.reshape(S, 128, D), axis=2))
            h = g + _SLOTS - _NC
            if g >= _NC and h < G and c < _NC - _HIN:
                out_copy(g - _NC).wait()
                in_copy(h).start()

        for c in range(_NC - _HIN, _NC):
            g = b * _NC + c
            h = g + _SLOTS - _NC
            if g >= _NC and h < G:
                out_copy(g - _NC).wait()
                in_copy(h).start()

        bits = jax.lax.bitcast_convert_type(nrm_c[...], jnp.int32)

        def srch(_, lh, bits=bits):
            lo, hi = lh
            delta = hi - lo
            q, r = delta // 8, delta % 8
            jj = jax.lax.broadcasted_iota(jnp.int32, (1, 2, 7), 2) + 1
            m = (lo[:, :, None] + q[:, :, None] * jj
                 + (r[:, :, None] * jj) // 8)
            cnt = jnp.sum((bits[None, None, None, :, :]
                           <= m[:, :, :, None, None]).astype(jnp.int32),
                          axis=(3, 4))
            t3 = targets[:, :, None]
            big = jnp.full_like(m, 0x7F800001)
            new_lo = jnp.maximum(lo, jnp.max(
                jnp.where(cnt < t3, m + 1, 0), axis=2))
            new_hi = jnp.minimum(hi, jnp.min(
                jnp.where(cnt >= t3, m, big), axis=2))
            return new_lo, new_hi

        lo0 = jnp.zeros((1, 2), jnp.int32)
        hi0 = jnp.full((1, 2), 0x7F800000, jnp.int32)
        lo, _ = jax.lax.fori_loop(0, 13, srch, (lo0, hi0))
        v = jax.lax.bitcast_convert_type(lo, jnp.float32)
        thres = v[:, 0:1] * 0.5 + v[:, 1:2] * 0.5

        for c in range(_NC):
            g = b * _NC + c
            m = (~(nrm_k[pl.ds(c * R, R), :] < thres)).astype(jnp.float32)
            buf[g % _SLOTS] = buf[g % _SLOTS] * m
            out_copy(g).start()

    for g in range(max(G - _SLOTS, 0), G):
        out_copy(g).wait()


def kernel(x):
    B, N, D = x.shape
    k_lo = int(_Q * (N - 1))
    k_hi = k_lo + (1 if _Q * (N - 1) != k_lo else 0)
    R = N // _NC
    return pl.pallas_call(
        functools.partial(_fused_body, k_lo, k_hi, B, N, D),
        in_specs=[pl.BlockSpec(memory_space=pl.ANY)],
        out_specs=pl.BlockSpec(memory_space=pl.ANY),
        out_shape=jax.ShapeDtypeStruct((B, N, D), x.dtype),
        scratch_shapes=[
            pltpu.VMEM((_SLOTS, R, D), jnp.float32),
            pltpu.VMEM((N, 1), jnp.float32),
            pltpu.VMEM((N // 128, 128), jnp.float32),
            pltpu.SemaphoreType.DMA((_SLOTS,)),
            pltpu.SemaphoreType.DMA((_SLOTS,)),
        ],
    )(x)


# speedup vs baseline: 1.0527x; 1.0087x over previous
import functools

import jax
import jax.numpy as jnp
from jax.experimental import pallas as pl
from jax.experimental.pallas import tpu as pltpu

_Q = 0.5
_NC = 16
_SLOTS = 26
_HIN = 3


def _fused_body(k_lo, k_hi, B, N, D, x_hbm, o_hbm, buf, nrm_k, nrm_c,
                sem_in, sem_out):
    R = N // _NC
    S = R // 128
    G = B * _NC

    def in_copy(g):
        b, c = divmod(g, _NC)
        return pltpu.make_async_copy(
            x_hbm.at[b, pl.ds(c * R, R), :],
            buf.at[g % _SLOTS],
            sem_in.at[g % _SLOTS])

    def out_copy(g):
        b, c = divmod(g, _NC)
        return pltpu.make_async_copy(
            buf.at[g % _SLOTS],
            o_hbm.at[b, pl.ds(c * R, R), :],
            sem_out.at[g % _SLOTS])

    col = jax.lax.broadcasted_iota(jnp.int32, (1, 2), 1)
    targets = jnp.where(col == 0, k_lo + 1, k_hi + 1)

    for g in range(min(_SLOTS, G)):
        in_copy(g).start()

    for b in range(B):
        for c in range(_NC):
            g = b * _NC + c
            in_copy(g).wait()
            xb = buf[g % _SLOTS]
            sq = xb * xb
            nrm_k[pl.ds(c * R, R), :] = jnp.sqrt(
                jnp.sum(sq, axis=1, keepdims=True))
            nrm_c[pl.ds(c * S, S), :] = jnp.sqrt(
                jnp.sum(sq.reshape(S, 128, D), axis=2))
            h = g + _SLOTS - _NC
            if g >= _NC and h < G and c < _NC - _HIN:
                out_copy(g - _NC).wait()
                in_copy(h).start()

        for c in range(_NC - _HIN, _NC):
            g = b * _NC + c
            h = g + _SLOTS - _NC
            if g >= _NC and h < G:
                out_copy(g - _NC).wait()
                in_copy(h).start()

        bits = jax.lax.bitcast_convert_type(nrm_c[...], jnp.int32)

        def srch(_, lh, bits=bits):
            lo, hi = lh
            delta = hi - lo
            q, r = delta // 8, delta % 8
            jj = jax.lax.broadcasted_iota(jnp.int32, (1, 2, 7), 2) + 1
            m = (lo[:, :, None] + q[:, :, None] * jj
                 + (r[:, :, None] * jj) // 8)
            cnt = jnp.sum((bits[None, None, None, :, :]
                           <= m[:, :, :, None, None]).astype(jnp.int32),
                          axis=(3, 4))
            t3 = targets[:, :, None]
            big = jnp.full_like(m, 0x7F800001)
            new_lo = jnp.maximum(lo, jnp.max(
                jnp.where(cnt < t3, m + 1, 0), axis=2))
            new_hi = jnp.minimum(hi, jnp.min(
                jnp.where(cnt >= t3, m, big), axis=2))
            return new_lo, new_hi

        lo0 = jnp.zeros((1, 2), jnp.int32)
        hi0 = jnp.full((1, 2), 0x7F800000, jnp.int32)
        lh = (lo0, hi0)
        for _ in range(13):
            lh = srch(None, lh)
        lo, _ = lh
        v = jax.lax.bitcast_convert_type(lo, jnp.float32)
        thres = v[:, 0:1] * 0.5 + v[:, 1:2] * 0.5

        for c in range(_NC):
            g = b * _NC + c
            m = (~(nrm_k[pl.ds(c * R, R), :] < thres)).astype(jnp.float32)
            buf[g % _SLOTS] = buf[g % _SLOTS] * m
            out_copy(g).start()

    for g in range(max(G - _SLOTS, 0), G):
        out_copy(g).wait()


def kernel(x):
    B, N, D = x.shape
    k_lo = int(_Q * (N - 1))
    k_hi = k_lo + (1 if _Q * (N - 1) != k_lo else 0)
    R = N // _NC
    return pl.pallas_call(
        functools.partial(_fused_body, k_lo, k_hi, B, N, D),
        in_specs=[pl.BlockSpec(memory_space=pl.ANY)],
        out_specs=pl.BlockSpec(memory_space=pl.ANY),
        out_shape=jax.ShapeDtypeStruct((B, N, D), x.dtype),
        scratch_shapes=[
            pltpu.VMEM((_SLOTS, R, D), jnp.float32),
            pltpu.VMEM((N, 1), jnp.float32),
            pltpu.VMEM((N // 128, 128), jnp.float32),
            pltpu.SemaphoreType.DMA((_SLOTS,)),
            pltpu.SemaphoreType.DMA((_SLOTS,)),
        ],
    )(x)
